# Initial kernel scaffold; baseline (speedup 1.0000x reference)
import jax
import jax.numpy as jnp
from jax.experimental import pallas as pl


def kernel(x, edge_index, W1a, b1a, W1b, b1b, W2a, b2a, W2b, b2b):
    raise NotImplementedError("write your pallas kernel here")


# speedup vs baseline: 3.0261x; 3.0261x over previous
import functools

import jax
import jax.numpy as jnp
from jax import lax
from jax.experimental import pallas as pl
from jax.experimental.pallas import tpu as pltpu
from jax.experimental.pallas import tpu_sc as plsc

N_NODES = 10000
D_IN = 256
HALF = 128
NC = 2
NS = 16
L = 16
CHUNK = 128
NSH = N_NODES + 112
GARBAGE_ROW = N_NODES


def _sc_segment_sum(table2, srcp, dstp, zrows):
    e_pad = srcp.shape[0]
    ept = e_pad // NS
    nch = ept // CHUNK
    rpt = NSH // NS

    mesh = plsc.VectorSubcoreMesh(core_axis_name="c", subcore_axis_name="s")

    @functools.partial(
        pl.kernel,
        out_type=jax.ShapeDtypeStruct((NC * NSH, HALF), jnp.float32),
        mesh=mesh,
        scratch_types=[
            pltpu.VMEM((CHUNK,), jnp.int32),
            pltpu.VMEM((CHUNK,), jnp.int32),
            pltpu.VMEM((CHUNK, HALF), jnp.float32),
            pltpu.VMEM_SHARED((NSH, HALF), jnp.float32),
            pltpu.SemaphoreType.DMA,
        ],
    )
    def k(table_h, src_h, dst_h, z_h, out_h, srcv, dstv, rows, acc, sem):
        c = lax.axis_index("c")
        s = lax.axis_index("s")
        r0 = s * rpt
        pltpu.sync_copy(z_h.at[pl.ds(r0, rpt)], acc.at[pl.ds(r0, rpt)])
        plsc.subcore_barrier()

        base = s * ept

        def body(j, carry):
            e0 = base + j * CHUNK
            pltpu.sync_copy(src_h.at[pl.ds(e0, CHUNK)], srcv)
            for i in range(CHUNK // L):
                sl = pl.ds(i * L, L)
                srcv[sl] = srcv[sl] * 2 + c
            pltpu.sync_copy(dst_h.at[pl.ds(e0, CHUNK)], dstv)
            pltpu.async_copy(table_h.at[srcv], rows, sem).wait()
            pltpu.sync_copy(rows, acc.at[dstv], add=True)
            return carry

        lax.fori_loop(0, nch, body, 0)
        plsc.subcore_barrier()
        pltpu.sync_copy(acc.at[pl.ds(r0, rpt)],
                        out_h.at[pl.ds(c * NSH + r0, rpt)])

    return k(table2, srcp, dstp, zrows)


def _tc_mlp(xv, agg, Wa, ba, Wb, bb, relu_out, d_out, rows_per_block):
    n = xv.shape[0]
    grid = n // rows_per_block
    r = rows_per_block
    split_out = d_out == D_IN

    def body(x_ref, a_ref, wa_ref, ba_ref, wb_ref, bb_ref, o_ref):
        hin = jnp.concatenate(
            [x_ref[:, 0, :] + a_ref[0], x_ref[:, 1, :] + a_ref[1]], axis=1)
        t = jnp.dot(hin, wa_ref[...], preferred_element_type=jnp.float32)
        t = jnp.maximum(t + ba_ref[...], 0.0)
        h = jnp.dot(t, wb_ref[...], preferred_element_type=jnp.float32)
        h = h + bb_ref[...]
        if relu_out:
            h = jnp.maximum(h, 0.0)
        if split_out:
            o_ref[:, 0, :] = h[:, :HALF]
            o_ref[:, 1, :] = h[:, HALF:]
        else:
            o_ref[...] = h

    if split_out:
        out_shape = jax.ShapeDtypeStruct((n, 2, HALF), jnp.float32)
        out_spec = pl.BlockSpec((r, 2, HALF), lambda i: (i, 0, 0))
    else:
        out_shape = jax.ShapeDtypeStruct((n, d_out), jnp.float32)
        out_spec = pl.BlockSpec((r, d_out), lambda i: (i, 0))

    return pl.pallas_call(
        body,
        grid=(grid,),
        in_specs=[
            pl.BlockSpec((r, 2, HALF), lambda i: (i, 0, 0)),
            pl.BlockSpec((2, r, HALF), lambda i: (0, i, 0)),
            pl.BlockSpec(Wa.shape, lambda i: (0, 0)),
            pl.BlockSpec((1, D_IN), lambda i: (0, 0)),
            pl.BlockSpec(Wb.shape, lambda i: (0, 0)),
            pl.BlockSpec((1, d_out), lambda i: (0, 0)),
        ],
        out_specs=out_spec,
        out_shape=out_shape,
    )(xv, agg, Wa, ba.reshape(1, D_IN), Wb, bb.reshape(1, d_out))


def kernel(x, edge_index, W1a, b1a, W1b, b1b, W2a, b2a, W2b, b2b):
    n = x.shape[0]
    e = edge_index.shape[1]
    src = edge_index[0].astype(jnp.int32)
    dst = edge_index[1].astype(jnp.int32)

    step = NS * CHUNK
    e_pad = ((e + step - 1) // step) * step
    pad = e_pad - e
    srcp = jnp.concatenate([src, jnp.zeros((pad,), jnp.int32)])
    dstp = jnp.concatenate([dst, jnp.full((pad,), GARBAGE_ROW, jnp.int32)])
    zrows = jnp.zeros((NSH, HALF), jnp.float32)

    xv = x.reshape(n, 2, HALF)

    agg1 = _sc_segment_sum(xv.reshape(2 * n, HALF), srcp, dstp, zrows)
    agg1 = agg1.reshape(NC, NSH, HALF)
    h = _tc_mlp(xv, agg1, W1a, b1a, W1b, b1b,
                relu_out=True, d_out=D_IN, rows_per_block=1000)

    agg2 = _sc_segment_sum(h.reshape(2 * n, HALF), srcp, dstp, zrows)
    agg2 = agg2.reshape(NC, NSH, HALF)
    out = _tc_mlp(h, agg2, W2a, b2a, W2b, b2b,
                  relu_out=False, d_out=64, rows_per_block=1000)
    return out
